# baseline (speedup 1.0000x reference)
import functools

import jax
import jax.numpy as jnp
from jax import lax
from jax.experimental import pallas as pl
from jax.experimental.pallas import tpu as pltpu
from jax.experimental.pallas import tpu_sc as plsc

F32 = jnp.float32

N_WORKERS = 32
GATHER_CHUNK = 80
SCATTER_CHUNK = 80
EDGE_BLOCK = 3200


def _pq_body(x_ref, s_ref, r_ref, be1_ref, p_ref, q_ref):
    x = x_ref[...]
    p = jnp.dot(x, s_ref[...], preferred_element_type=F32) + be1_ref[0:1, :]
    q = jnp.dot(x, r_ref[...], preferred_element_type=F32)
    p_ref[...] = p.astype(jnp.bfloat16)
    q_ref[...] = q.astype(jnp.bfloat16)


N_SLOTS = 5


@functools.cache
def _make_gather_kernel(E, D, chunk):
    e_per_w = E // N_WORKERS
    n_chunks = e_per_w // chunk
    n_outer = n_chunks // N_SLOTS
    mesh = plsc.VectorSubcoreMesh(core_axis_name="c", subcore_axis_name="s")

    @functools.partial(
        pl.kernel,
        out_type=jax.ShapeDtypeStruct((E, D), F32),
        mesh=mesh,
        scratch_types=(
            [pltpu.VMEM((n_chunks, chunk), jnp.int32)] * 2
            + [pltpu.VMEM((chunk, D), jnp.bfloat16)] * (2 * N_SLOTS)
            + [pltpu.VMEM((chunk, D), F32)] * N_SLOTS
            + [pltpu.SemaphoreType.DMA] * (2 * N_SLOTS + 1)
        ),
        compiler_params=pltpu.CompilerParams(
            use_tc_tiling_on_sc=False, needs_layout_passes=False
        ),
    )
    def gather_kernel(p_hbm, q_hbm, src_hbm, dst_hbm, g_hbm, *bufs):
        si_all, di_all = bufs[0], bufs[1]
        ga = bufs[2 : 2 + N_SLOTS]
        gb = bufs[2 + N_SLOTS : 2 + 2 * N_SLOTS]
        gc = bufs[2 + 2 * N_SLOTS : 2 + 3 * N_SLOTS]
        sem_g = bufs[2 + 3 * N_SLOTS : 2 + 4 * N_SLOTS]
        sem_st = bufs[2 + 4 * N_SLOTS : 2 + 5 * N_SLOTS]
        sem_idx = bufs[2 + 5 * N_SLOTS]

        wid = lax.axis_index("s") * 2 + lax.axis_index("c")
        base_w = wid * e_per_w

        di = pltpu.async_copy(src_hbm.at[pl.ds(wid * n_chunks, n_chunks)], si_all, sem_idx)
        dd = pltpu.async_copy(dst_hbm.at[pl.ds(wid * n_chunks, n_chunks)], di_all, sem_idx)
        di.wait()
        dd.wait()

        def outer(j, carry):
            descs = []
            for s in range(N_SLOTS):
                i = j * N_SLOTS + s

                @pl.when(j > 0)
                def _(s=s):
                    pltpu.make_async_copy(
                        gc[s], g_hbm.at[pl.ds(0, chunk)], sem_st[s]
                    ).wait()

                descs.append((
                    pltpu.async_copy(p_hbm.at[si_all.at[i]], ga[s], sem_g[s]),
                    pltpu.async_copy(q_hbm.at[di_all.at[i]], gb[s], sem_g[s]),
                ))
            for s in range(N_SLOTS):
                i = j * N_SLOTS + s
                descs[s][0].wait()
                descs[s][1].wait()
                ga_s, gb_s, gc_s = ga[s], gb[s], gc[s]

                def add_body(e, c, ga_s=ga_s, gb_s=gb_s, gc_s=gc_s):
                    for jj in range(D // 32):
                        sl = pl.ds(jj * 32, 32)
                        v = ga_s[e, sl] + gb_s[e, sl]
                        lo, hi = plsc.unpack(v, format=plsc.PackFormat.INTERLEAVED)
                        gc_s[e, pl.ds(jj * 16, 16)] = lo
                        gc_s[e, pl.ds(D // 2 + jj * 16, 16)] = hi
                    return c

                lax.fori_loop(0, chunk, add_body, 0)
                pltpu.async_copy(
                    gc[s], g_hbm.at[pl.ds(base_w + i * chunk, chunk)], sem_st[s]
                )
            return carry

        lax.fori_loop(0, n_outer, outer, 0)
        for s in range(N_SLOTS):
            pltpu.make_async_copy(gc[s], g_hbm.at[pl.ds(0, chunk)], sem_st[s]).wait()

    return gather_kernel


def _edge_body(g_ref, eat_ref, a_ref, we2_ref, be2_ref, out_ref):
    t = jax.lax.dot_general(
        eat_ref[...], a_ref[...], (((0,), (0,)), ((), ())),
        preferred_element_type=F32,
    )
    h = jnp.maximum(g_ref[...] + t, 0.0)
    out_ref[...] = jnp.dot(h, we2_ref[...], preferred_element_type=F32) + be2_ref[0:1, :]


@functools.cache
def _make_scatter_kernel(E, N, De, chunk):
    e_per_w = E // N_WORKERS
    n_chunks = e_per_w // chunk
    n_stripes = 10
    stripe = N // n_stripes
    mesh = plsc.VectorSubcoreMesh(core_axis_name="c", subcore_axis_name="s")

    @functools.partial(
        pl.kernel,
        out_type=jax.ShapeDtypeStruct((2 * N, De), F32),
        mesh=mesh,
        scratch_types=(
            [
                pltpu.VMEM((n_chunks, chunk), jnp.int32),
                pltpu.VMEM((stripe, De), F32),
                pltpu.VMEM_SHARED((N, De), F32),
                pltpu.SemaphoreType.DMA,
            ]
            + [pltpu.VMEM((chunk, De), F32)] * N_SLOTS
            + [pltpu.SemaphoreType.DMA] * (2 * N_SLOTS)
        ),
        compiler_params=pltpu.CompilerParams(use_tc_tiling_on_sc=False),
    )
    def scatter_kernel(e_hbm, dst_hbm, out_hbm, *bufs):
        di_all, z_v, agg_sh, sem_idx = bufs[0], bufs[1], bufs[2], bufs[3]
        ev = bufs[4 : 4 + N_SLOTS]
        sem_ld = bufs[4 + N_SLOTS : 4 + 2 * N_SLOTS]
        sem_sc = bufs[4 + 2 * N_SLOTS : 4 + 3 * N_SLOTS]

        cid = lax.axis_index("c")
        sid = lax.axis_index("s")
        wid = sid * 2 + cid
        base_w = wid * e_per_w

        d_idx = pltpu.async_copy(
            dst_hbm.at[pl.ds(wid * n_chunks, n_chunks)], di_all, sem_idx
        )

        def zero_body(j, c):
            z_v[j, :] = jnp.zeros((De,), F32)
            return c

        lax.fori_loop(0, stripe, zero_body, 0)

        @pl.when(sid < n_stripes)
        def _():
            pltpu.sync_copy(z_v, agg_sh.at[pl.ds(sid * stripe, stripe)])

        d_idx.wait()
        plsc.subcore_barrier()

        n_outer = n_chunks // N_SLOTS

        def outer(j, carry):
            descs = []
            for s in range(N_SLOTS):
                i = j * N_SLOTS + s

                @pl.when(j > 0)
                def _(s=s):
                    pltpu.make_async_copy(
                        ev[s], agg_sh.at[di_all.at[0]], sem_sc[s]
                    ).wait()

                descs.append(
                    pltpu.async_copy(
                        e_hbm.at[pl.ds(base_w + i * chunk, chunk), pl.ds(0, De)],
                        ev[s],
                        sem_ld[s],
                    )
                )
            for s in range(N_SLOTS):
                i = j * N_SLOTS + s
                descs[s].wait()
                pltpu.async_copy(ev[s], agg_sh.at[di_all.at[i]], sem_sc[s], add=True)
            return carry

        lax.fori_loop(0, n_outer, outer, 0)
        for s in range(N_SLOTS):
            pltpu.make_async_copy(ev[s], agg_sh.at[di_all.at[0]], sem_sc[s]).wait()
        plsc.subcore_barrier()

        @pl.when(sid < n_stripes)
        def _():
            pltpu.sync_copy(
                agg_sh.at[pl.ds(sid * stripe, stripe)],
                out_hbm.at[pl.ds(cid * N + sid * stripe, stripe)],
            )

    return scatter_kernel


def _node_body(x_ref, *refs):
    aggp_refs = refs[:-6]
    w1x_ref, w1a_ref, bn1_ref, w2_ref, bn2_ref, out_ref = refs[-6:]
    agg = aggp_refs[0][...] + aggp_refs[1][...]
    for r in aggp_refs[2:]:
        agg = agg + r[...]
    h = (
        jnp.dot(x_ref[...], w1x_ref[...], preferred_element_type=F32)
        + jnp.dot(agg, w1a_ref[...], preferred_element_type=F32)
        + bn1_ref[0:1, :]
    )
    h = jnp.maximum(h, 0.0)
    out_ref[...] = jnp.dot(h, w2_ref[...], preferred_element_type=F32) + bn2_ref[0:1, :]


def kernel(x, edge_index, edge_attr, We1, be1, We2, be2, Wn1, bn1, Wn2, bn2):
    N, Dn = x.shape
    E = edge_index.shape[1]
    De = edge_attr.shape[1]
    H = We1.shape[1]

    A = We1[:De]
    S = We1[De : De + Dn]
    R = We1[De + Dn :]
    W1x = Wn1[:Dn]
    W1a = Wn1[Dn:]
    src = edge_index[0]
    dst = edge_index[1]

    be1_b = jnp.broadcast_to(be1.reshape(1, H), (8, H))
    be2_b = jnp.broadcast_to(be2.reshape(1, De), (8, De))
    bn1_b = jnp.broadcast_to(bn1.reshape(1, H), (8, H))
    bn2_b = jnp.broadcast_to(bn2.reshape(1, Dn), (8, Dn))

    perm = jnp.stack([jnp.arange(H // 2), jnp.arange(H // 2) + H // 2], axis=1).reshape(-1)
    P, Q = pl.pallas_call(
        _pq_body,
        out_shape=[
            jax.ShapeDtypeStruct((N, H), jnp.bfloat16),
            jax.ShapeDtypeStruct((N, H), jnp.bfloat16),
        ],
    )(x, S[:, perm], R[:, perm], be1_b[:, perm])

    seg_sizes = [166400, 153600]
    src2 = src.reshape(-1, GATHER_CHUNK)
    dst2 = dst.reshape(-1, GATHER_CHUNK)
    ea_t = edge_attr.T
    We2p = jnp.pad(We2, ((0, 0), (0, Dn - De)))
    be2p = jnp.broadcast_to(
        jnp.pad(be2, (0, Dn - De)).reshape(1, Dn), (8, Dn)
    )

    e_parts, agg_parts = [], []
    row0, blk0 = 0, 0
    for E_seg in seg_sizes:
        rows_seg = E_seg // GATHER_CHUNK
        n_blocks = E_seg // EDGE_BLOCK
        src_k = src2[row0 : row0 + rows_seg]
        dst_k = dst2[row0 : row0 + rows_seg]
        G = _make_gather_kernel(E_seg, H, GATHER_CHUNK)(P, Q, src_k, dst_k)
        e128 = pl.pallas_call(
            _edge_body,
            grid=(n_blocks,),
            in_specs=[
                pl.BlockSpec((EDGE_BLOCK, H), lambda i: (i, 0)),
                pl.BlockSpec(
                    (De, EDGE_BLOCK), lambda i, blk0=blk0: (0, i + blk0)
                ),
                pl.BlockSpec((De, H), lambda i: (0, 0)),
                pl.BlockSpec((H, Dn), lambda i: (0, 0)),
                pl.BlockSpec((8, Dn), lambda i: (0, 0)),
            ],
            out_specs=pl.BlockSpec((EDGE_BLOCK, Dn), lambda i: (i, 0)),
            out_shape=jax.ShapeDtypeStruct((E_seg, Dn), F32),
        )(G, ea_t, A, We2p, be2p)
        e_parts.append(e128[:, :De])
        agg_parts.append(_make_scatter_kernel(E_seg, N, De, SCATTER_CHUNK)(e128, dst_k))
        row0 += rows_seg
        blk0 += n_blocks
    e_new = jnp.concatenate(e_parts, axis=0)

    NODE_BLOCK = 1000
    n_node_blocks = N // NODE_BLOCK
    agg_in, agg_specs = [], []
    for a in agg_parts:
        agg_in += [a, a]
        agg_specs += [
            pl.BlockSpec((NODE_BLOCK, De), lambda i: (i, 0)),
            pl.BlockSpec((NODE_BLOCK, De), lambda i: (i + n_node_blocks, 0)),
        ]
    x_new = pl.pallas_call(
        _node_body,
        grid=(n_node_blocks,),
        in_specs=[pl.BlockSpec((NODE_BLOCK, Dn), lambda i: (i, 0))]
        + agg_specs
        + [
            pl.BlockSpec((Dn, H), lambda i: (0, 0)),
            pl.BlockSpec((De, H), lambda i: (0, 0)),
            pl.BlockSpec((8, H), lambda i: (0, 0)),
            pl.BlockSpec((H, Dn), lambda i: (0, 0)),
            pl.BlockSpec((8, Dn), lambda i: (0, 0)),
        ],
        out_specs=pl.BlockSpec((NODE_BLOCK, Dn), lambda i: (i, 0)),
        out_shape=jax.ShapeDtypeStruct((N, Dn), F32),
    )(x, *agg_in, W1x, W1a, bn1_b, Wn2, bn2_b)

    return x_new, e_new


# speedup vs baseline: 1.2282x; 1.2282x over previous
import functools

import jax
import jax.numpy as jnp
from jax import lax
from jax.experimental import pallas as pl
from jax.experimental.pallas import tpu as pltpu
from jax.experimental.pallas import tpu_sc as plsc

F32 = jnp.float32

N_WORKERS = 32
GATHER_CHUNK = 80
SCATTER_CHUNK = 80
EDGE_BLOCK = 6400


def _pq_body(x_ref, s_ref, r_ref, be1_ref, p_ref, q_ref):
    x = x_ref[...]
    p_ref[...] = jnp.dot(x, s_ref[...], preferred_element_type=F32) + be1_ref[0:1, :]
    q_ref[...] = jnp.dot(x, r_ref[...], preferred_element_type=F32)


N_SLOTS = 5


@functools.cache
def _make_gather_kernel(E, D, chunk):
    e_per_w = E // N_WORKERS
    n_chunks = e_per_w // chunk
    n_outer = n_chunks // N_SLOTS
    mesh = plsc.VectorSubcoreMesh(core_axis_name="c", subcore_axis_name="s")

    @functools.partial(
        pl.kernel,
        out_type=jax.ShapeDtypeStruct((E, D), F32),
        mesh=mesh,
        scratch_types=(
            [pltpu.VMEM((n_chunks, chunk), jnp.int32)] * 2
            + [pltpu.VMEM((chunk, D), F32)] * (2 * N_SLOTS)
            + [pltpu.SemaphoreType.DMA] * (2 * N_SLOTS + 1)
        ),
        compiler_params=pltpu.CompilerParams(use_tc_tiling_on_sc=False),
    )
    def gather_kernel(p_hbm, q_hbm, src_hbm, dst_hbm, g_hbm, *bufs):
        si_all, di_all = bufs[0], bufs[1]
        ga = bufs[2 : 2 + N_SLOTS]
        gb = bufs[2 + N_SLOTS : 2 + 2 * N_SLOTS]
        sem_g = bufs[2 + 2 * N_SLOTS : 2 + 3 * N_SLOTS]
        sem_st = bufs[2 + 3 * N_SLOTS : 2 + 4 * N_SLOTS]
        sem_idx = bufs[2 + 4 * N_SLOTS]

        wid = lax.axis_index("s") * 2 + lax.axis_index("c")
        base_w = wid * e_per_w

        di = pltpu.async_copy(src_hbm.at[pl.ds(wid * n_chunks, n_chunks)], si_all, sem_idx)
        dd = pltpu.async_copy(dst_hbm.at[pl.ds(wid * n_chunks, n_chunks)], di_all, sem_idx)
        di.wait()
        dd.wait()

        def outer(j, carry):
            descs = []
            for s in range(N_SLOTS):
                i = j * N_SLOTS + s

                @pl.when(j > 0)
                def _(s=s):
                    pltpu.make_async_copy(
                        ga[s], g_hbm.at[pl.ds(0, chunk)], sem_st[s]
                    ).wait()

                descs.append((
                    pltpu.async_copy(p_hbm.at[si_all.at[i]], ga[s], sem_g[s]),
                    pltpu.async_copy(q_hbm.at[di_all.at[i]], gb[s], sem_g[s]),
                ))
            for s in range(N_SLOTS):
                i = j * N_SLOTS + s
                descs[s][0].wait()
                descs[s][1].wait()
                ga_s, gb_s = ga[s], gb[s]

                def add_body(e, c, ga_s=ga_s, gb_s=gb_s):
                    for jj in range(D // 16):
                        sl = pl.ds(jj * 16, 16)
                        ga_s[e, sl] = ga_s[e, sl] + gb_s[e, sl]
                    return c

                lax.fori_loop(0, chunk, add_body, 0)
                pltpu.async_copy(
                    ga[s], g_hbm.at[pl.ds(base_w + i * chunk, chunk)], sem_st[s]
                )
            return carry

        lax.fori_loop(0, n_outer, outer, 0)
        for s in range(N_SLOTS):
            pltpu.make_async_copy(ga[s], g_hbm.at[pl.ds(0, chunk)], sem_st[s]).wait()

    return gather_kernel


def _edge_body(g_ref, eat_ref, a_ref, we2_ref, be2_ref, out_ref):
    t = jax.lax.dot_general(
        eat_ref[...], a_ref[...], (((0,), (0,)), ((), ())),
        preferred_element_type=F32,
    )
    h = jnp.maximum(g_ref[...] + t, 0.0)
    out_ref[...] = jnp.dot(h, we2_ref[...], preferred_element_type=F32) + be2_ref[0:1, :]


@functools.cache
def _make_scatter_kernel(E, N, De, chunk):
    e_per_w = E // N_WORKERS
    n_chunks = e_per_w // chunk
    n_stripes = 10
    stripe = N // n_stripes
    mesh = plsc.VectorSubcoreMesh(core_axis_name="c", subcore_axis_name="s")

    @functools.partial(
        pl.kernel,
        out_type=jax.ShapeDtypeStruct((2 * N, De), F32),
        mesh=mesh,
        scratch_types=(
            [
                pltpu.VMEM((n_chunks, chunk), jnp.int32),
                pltpu.VMEM((stripe, De), F32),
                pltpu.VMEM_SHARED((N, De), F32),
                pltpu.SemaphoreType.DMA,
            ]
            + [pltpu.VMEM((chunk, De), F32)] * N_SLOTS
            + [pltpu.SemaphoreType.DMA] * (2 * N_SLOTS)
        ),
        compiler_params=pltpu.CompilerParams(use_tc_tiling_on_sc=False),
    )
    def scatter_kernel(e_hbm, dst_hbm, out_hbm, *bufs):
        di_all, z_v, agg_sh, sem_idx = bufs[0], bufs[1], bufs[2], bufs[3]
        ev = bufs[4 : 4 + N_SLOTS]
        sem_ld = bufs[4 + N_SLOTS : 4 + 2 * N_SLOTS]
        sem_sc = bufs[4 + 2 * N_SLOTS : 4 + 3 * N_SLOTS]

        cid = lax.axis_index("c")
        sid = lax.axis_index("s")
        wid = sid * 2 + cid
        base_w = wid * e_per_w

        d_idx = pltpu.async_copy(
            dst_hbm.at[pl.ds(wid * n_chunks, n_chunks)], di_all, sem_idx
        )

        def zero_body(j, c):
            z_v[j, :] = jnp.zeros((De,), F32)
            return c

        lax.fori_loop(0, stripe, zero_body, 0)

        @pl.when(sid < n_stripes)
        def _():
            pltpu.sync_copy(z_v, agg_sh.at[pl.ds(sid * stripe, stripe)])

        d_idx.wait()
        plsc.subcore_barrier()

        n_outer = n_chunks // N_SLOTS

        def outer(j, carry):
            descs = []
            for s in range(N_SLOTS):
                i = j * N_SLOTS + s

                @pl.when(j > 0)
                def _(s=s):
                    pltpu.make_async_copy(
                        ev[s], agg_sh.at[di_all.at[0]], sem_sc[s]
                    ).wait()

                descs.append(
                    pltpu.async_copy(
                        e_hbm.at[pl.ds(base_w + i * chunk, chunk), pl.ds(0, De)],
                        ev[s],
                        sem_ld[s],
                    )
                )
            for s in range(N_SLOTS):
                i = j * N_SLOTS + s
                descs[s].wait()
                pltpu.async_copy(ev[s], agg_sh.at[di_all.at[i]], sem_sc[s], add=True)
            return carry

        lax.fori_loop(0, n_outer, outer, 0)
        for s in range(N_SLOTS):
            pltpu.make_async_copy(ev[s], agg_sh.at[di_all.at[0]], sem_sc[s]).wait()
        plsc.subcore_barrier()

        @pl.when(sid < n_stripes)
        def _():
            pltpu.sync_copy(
                agg_sh.at[pl.ds(sid * stripe, stripe)],
                out_hbm.at[pl.ds(cid * N + sid * stripe, stripe)],
            )

    return scatter_kernel


def _node_body(x_ref, *refs):
    aggp_refs = refs[:-6]
    w1x_ref, w1a_ref, bn1_ref, w2_ref, bn2_ref, out_ref = refs[-6:]
    agg = aggp_refs[0][...] + aggp_refs[1][...]
    for r in aggp_refs[2:]:
        agg = agg + r[...]
    h = (
        jnp.dot(x_ref[...], w1x_ref[...], preferred_element_type=F32)
        + jnp.dot(agg, w1a_ref[...], preferred_element_type=F32)
        + bn1_ref[0:1, :]
    )
    h = jnp.maximum(h, 0.0)
    out_ref[...] = jnp.dot(h, w2_ref[...], preferred_element_type=F32) + bn2_ref[0:1, :]


def kernel(x, edge_index, edge_attr, We1, be1, We2, be2, Wn1, bn1, Wn2, bn2):
    N, Dn = x.shape
    E = edge_index.shape[1]
    De = edge_attr.shape[1]
    H = We1.shape[1]

    A = We1[:De]
    S = We1[De : De + Dn]
    R = We1[De + Dn :]
    W1x = Wn1[:Dn]
    W1a = Wn1[Dn:]
    src = edge_index[0]
    dst = edge_index[1]

    be1_b = jnp.broadcast_to(be1.reshape(1, H), (8, H))
    be2_b = jnp.broadcast_to(be2.reshape(1, De), (8, De))
    bn1_b = jnp.broadcast_to(bn1.reshape(1, H), (8, H))
    bn2_b = jnp.broadcast_to(bn2.reshape(1, Dn), (8, Dn))

    P, Q = pl.pallas_call(
        _pq_body,
        out_shape=[
            jax.ShapeDtypeStruct((N, H), F32),
            jax.ShapeDtypeStruct((N, H), F32),
        ],
    )(x, S, R, be1_b)

    seg_sizes = [166400, 153600]
    src2 = src.reshape(-1, GATHER_CHUNK)
    dst2 = dst.reshape(-1, GATHER_CHUNK)
    ea_t = edge_attr.T
    We2p = jnp.pad(We2, ((0, 0), (0, Dn - De)))
    be2p = jnp.broadcast_to(
        jnp.pad(be2, (0, Dn - De)).reshape(1, Dn), (8, Dn)
    )

    e_parts, agg_parts = [], []
    row0, blk0 = 0, 0
    for E_seg in seg_sizes:
        rows_seg = E_seg // GATHER_CHUNK
        n_blocks = E_seg // EDGE_BLOCK
        src_k = src2[row0 : row0 + rows_seg]
        dst_k = dst2[row0 : row0 + rows_seg]
        G = _make_gather_kernel(E_seg, H, GATHER_CHUNK)(P, Q, src_k, dst_k)
        e128 = pl.pallas_call(
            _edge_body,
            grid=(n_blocks,),
            in_specs=[
                pl.BlockSpec((EDGE_BLOCK, H), lambda i: (i, 0)),
                pl.BlockSpec(
                    (De, EDGE_BLOCK), lambda i, blk0=blk0: (0, i + blk0)
                ),
                pl.BlockSpec((De, H), lambda i: (0, 0)),
                pl.BlockSpec((H, Dn), lambda i: (0, 0)),
                pl.BlockSpec((8, Dn), lambda i: (0, 0)),
            ],
            out_specs=pl.BlockSpec((EDGE_BLOCK, Dn), lambda i: (i, 0)),
            out_shape=jax.ShapeDtypeStruct((E_seg, Dn), F32),
        )(G, ea_t, A, We2p, be2p)
        e_parts.append(e128[:, :De])
        agg_parts.append(_make_scatter_kernel(E_seg, N, De, SCATTER_CHUNK)(e128, dst_k))
        row0 += rows_seg
        blk0 += n_blocks
    e_new = jnp.concatenate(e_parts, axis=0)

    NODE_BLOCK = 1000
    n_node_blocks = N // NODE_BLOCK
    agg_in, agg_specs = [], []
    for a in agg_parts:
        agg_in += [a, a]
        agg_specs += [
            pl.BlockSpec((NODE_BLOCK, De), lambda i: (i, 0)),
            pl.BlockSpec((NODE_BLOCK, De), lambda i: (i + n_node_blocks, 0)),
        ]
    x_new = pl.pallas_call(
        _node_body,
        grid=(n_node_blocks,),
        in_specs=[pl.BlockSpec((NODE_BLOCK, Dn), lambda i: (i, 0))]
        + agg_specs
        + [
            pl.BlockSpec((Dn, H), lambda i: (0, 0)),
            pl.BlockSpec((De, H), lambda i: (0, 0)),
            pl.BlockSpec((8, H), lambda i: (0, 0)),
            pl.BlockSpec((H, Dn), lambda i: (0, 0)),
            pl.BlockSpec((8, Dn), lambda i: (0, 0)),
        ],
        out_specs=pl.BlockSpec((NODE_BLOCK, Dn), lambda i: (i, 0)),
        out_shape=jax.ShapeDtypeStruct((N, Dn), F32),
    )(x, *agg_in, W1x, W1a, bn1_b, Wn2, bn2_b)

    return x_new, e_new
